# baseline (speedup 1.0000x reference)
import functools

import jax
import jax.numpy as jnp
from jax import lax
from jax.experimental import pallas as pl
from jax.experimental.pallas import tpu as pltpu
from jax.experimental.pallas import tpu_sc as plsc

B = 1024
N = 100000
D = 64
C = 3000
TEMP = 0.05

NC = 2
NS = 16
NW = NC * NS
CHUNK = 128
CPW = 25
FULLC = 24 * NW + 13
TAIL0 = N - CHUNK
C_PAD = 3072
CNT_W = 16
STRIPE = C_PAD // NS
TB = B // NW

NBUF = 8
DEPTH = 4

BB = 256
GRID = B // BB


_sc_mesh = plsc.VectorSubcoreMesh(core_axis_name="c", subcore_axis_name="s")


@functools.partial(
    pl.kernel,
    out_type=[
        jax.ShapeDtypeStruct((NC, C_PAD, D), jnp.float32),
        jax.ShapeDtypeStruct((NC, C_PAD, CNT_W), jnp.float32),
        jax.ShapeDtypeStruct((B,), jnp.int32),
    ],
    mesh=_sc_mesh,
    scratch_types=[
        pltpu.VMEM((CPW * CHUNK,), jnp.int32),
        pltpu.VMEM((NBUF, CHUNK, D), jnp.float32),
        pltpu.VMEM((CHUNK, CNT_W), jnp.float32),
        pltpu.VMEM((STRIPE, D), jnp.float32),
        pltpu.VMEM((STRIPE, CNT_W), jnp.float32),
        pltpu.VMEM((TB,), jnp.int32),
        pltpu.VMEM((TB,), jnp.int32),
        pltpu.SemaphoreType.DMA,
        pltpu.SemaphoreType.DMA,
        pltpu.SemaphoreType.DMA,
    ] + [pltpu.SemaphoreType.DMA] * NBUF
      + [pltpu.SemaphoreType.DMA] * NBUF
      + [
        pltpu.VMEM_SHARED((C_PAD, D), jnp.float32),
        pltpu.VMEM_SHARED((C_PAD, CNT_W), jnp.float32),
    ],
    compiler_params=pltpu.CompilerParams(use_tc_tiling_on_sc=False),
)
def _sc_segment_sum(feat_hbm, lbl_hbm, idxs_hbm, featout, cntout, tgtout,
                    lbl_v, feat_v, ones_v, zf_v, zc_v, tidx_v, tgt_v,
                    sem_l, sem_c, sem_t, *rest):
    sem_ld = rest[:NBUF]
    sem_sc = rest[NBUF:2 * NBUF]
    acc_f, acc_c = rest[2 * NBUF:]
    cid = lax.axis_index("c")
    sid = lax.axis_index("s")
    wid = sid * NC + cid
    iota16 = lax.iota(jnp.int32, 16)
    zero16 = jnp.zeros((16,), jnp.float32)
    one0 = jnp.where(iota16 == 0, 1.0, 0.0).astype(jnp.float32)

    def chunk_rows(j):
        if j == CPW - 1:
            return pl.ds(jnp.minimum((24 * NW + wid) * CHUNK, TAIL0), CHUNK)
        return pl.ds((wid + j * NW) * CHUNK, CHUNK)

    def lbl_idx(j):
        return lbl_v.at[pl.ds(j * CHUNK, CHUNK)]

    lbl_loads = [
        pltpu.async_copy(lbl_hbm.at[chunk_rows(j)],
                         lbl_v.at[pl.ds(j * CHUNK, CHUNK)], sem_l)
        for j in range(CPW)
    ]
    load_h = [None] * CPW
    for j in range(DEPTH):
        load_h[j] = pltpu.async_copy(
            feat_hbm.at[chunk_rows(j)], feat_v.at[j % NBUF],
            sem_ld[j % NBUF])

    def _fill(r, carry):
        for c4 in range(D // 16):
            zf_v[r, pl.ds(c4 * 16, 16)] = zero16
        zc_v[r, pl.ds(0, 16)] = zero16
        return carry
    lax.fori_loop(0, STRIPE, _fill, 0)

    def _fill_ones(r, carry):
        ones_v[r, pl.ds(0, 16)] = one0
        return carry
    lax.fori_loop(0, CHUNK, _fill_ones, 0)

    row0 = sid * STRIPE
    pltpu.sync_copy(zf_v, acc_f.at[pl.ds(row0, STRIPE)])
    pltpu.sync_copy(zc_v, acc_c.at[pl.ds(row0, STRIPE)])
    plsc.subcore_barrier()

    tb0 = wid * TB
    pltpu.sync_copy(idxs_hbm.at[pl.ds(tb0, TB)], tidx_v)
    tgt_gather = pltpu.async_copy(lbl_hbm.at[tidx_v], tgt_v, sem_t)

    for h in lbl_loads:
        h.wait()

    t0 = (CPW - 1) * CHUNK
    for k in range(CHUNK // 16):
        thr = 14 if k >= 6 else 13
        s = jnp.minimum(jnp.maximum(thr - wid, 0), 1)
        v = lbl_v[pl.ds(t0 + k * 16, 16)]
        lbl_v[pl.ds(t0 + k * 16, 16)] = v * s + C * (1 - s)

    scat_h = [None] * CPW
    scat_done = [False] * CPW
    cnt_h = []
    for j in range(CPW):
        load_h[j].wait()
        scat_h[j] = pltpu.async_copy(
            feat_v.at[j % NBUF], acc_f.at[lbl_idx(j)],
            sem_sc[j % NBUF], add=True)
        cnt_h.append(
            pltpu.async_copy(ones_v, acc_c.at[lbl_idx(j)], sem_c, add=True))
        nxt = j + DEPTH
        if nxt < CPW:
            prev = nxt - NBUF
            if prev >= 0:
                scat_h[prev].wait()
                scat_done[prev] = True
            load_h[nxt] = pltpu.async_copy(
                feat_hbm.at[chunk_rows(nxt)], feat_v.at[nxt % NBUF],
                sem_ld[nxt % NBUF])
    for j in range(CPW):
        if not scat_done[j]:
            scat_h[j].wait()
    for h in cnt_h:
        h.wait()

    tgt_gather.wait()
    pltpu.sync_copy(tgt_v, tgtout.at[pl.ds(tb0, TB)])

    plsc.subcore_barrier()
    pltpu.sync_copy(acc_f.at[pl.ds(row0, STRIPE)],
                    featout.at[cid, pl.ds(row0, STRIPE)])
    pltpu.sync_copy(acc_c.at[pl.ds(row0, STRIPE)],
                    cntout.at[cid, pl.ds(row0, STRIPE)])


def _tc_body(x_ref, fp_ref, cp_ref, t_ref, out_ref):
    i = pl.program_id(0)
    fp = fp_ref[...]
    cs = fp[0] + fp[1]
    cp = cp_ref[...]
    cnt = jnp.sum(cp[0] + cp[1], axis=1, keepdims=True)
    rowid = lax.broadcasted_iota(jnp.int32, (C_PAD, 1), 0)
    nonempty = cnt > 0.0
    valid = jnp.logical_and(nonempty, rowid < C)
    denom = jnp.where(nonempty, cnt, 1.0)
    csw = cs * ((1.0 / TEMP) / denom)
    x = x_ref[...]
    vt = lax.dot_general(csw, x, (((1,), (1,)), ((), ())),
                         preferred_element_type=jnp.float32)
    et = jnp.exp(vt) * valid.astype(jnp.float32)
    s = jnp.sum(et, axis=0, keepdims=True)
    t = t_ref[0]
    cid2 = lax.broadcasted_iota(jnp.int32, (C_PAD, BB), 0)
    oh = (cid2 == t).astype(jnp.float32)
    val_t = jnp.sum(et * oh, axis=0, keepdims=True)
    lp = jnp.log(val_t / (s + 1e-6) + 1e-6)
    part = jnp.sum(lp, axis=1, keepdims=True)

    @pl.when(i == 0)
    def _():
        out_ref[...] = jnp.zeros_like(out_ref)

    out_ref[...] += part

    @pl.when(i == GRID - 1)
    def _():
        out_ref[...] = out_ref[...] * (-1.0 / B)


def _tc_loss(x, fp, cp, t3):
    return pl.pallas_call(
        _tc_body,
        grid=(GRID,),
        in_specs=[
            pl.BlockSpec((BB, D), lambda i: (i, 0)),
            pl.BlockSpec((NC, C_PAD, D), lambda i: (0, 0, 0)),
            pl.BlockSpec((NC, C_PAD, CNT_W), lambda i: (0, 0, 0)),
            pl.BlockSpec((1, 1, BB), lambda i: (i, 0, 0)),
        ],
        out_specs=pl.BlockSpec((1, 1), lambda i: (0, 0)),
        out_shape=jax.ShapeDtypeStruct((1, 1), jnp.float32),
    )(x, fp, cp, t3)


def kernel(inputs, indexes, features, labels, domain=0):
    feat_p, cnt_p, targets = _sc_segment_sum(features, labels, indexes)
    t3 = jnp.reshape(targets, (GRID, 1, BB))
    loss = _tc_loss(inputs, feat_p, cnt_p, t3)
    return jnp.reshape(loss, ())


# speedup vs baseline: 1.0769x; 1.0769x over previous
import functools

import jax
import jax.numpy as jnp
from jax import lax
from jax.experimental import pallas as pl
from jax.experimental.pallas import tpu as pltpu
from jax.experimental.pallas import tpu_sc as plsc

B = 1024
N = 100000
D = 64
DP = 128
C = 3000
TEMP = 0.05

NC = 2
NS = 16
NW = NC * NS
CHUNK = 128
CPW = 25
NP = NW * CPW * CHUNK
C_PAD = 3072
CNT_W = 16
STRIPE = C_PAD // NS
TB = B // NW

NBUF = 8
DEPTH = 4

BB = 256
GRID = B // BB


_sc_mesh = plsc.VectorSubcoreMesh(core_axis_name="c", subcore_axis_name="s")


@functools.partial(
    pl.kernel,
    out_type=[
        jax.ShapeDtypeStruct((NC, C_PAD, D), jnp.float32),
        jax.ShapeDtypeStruct((NC, C_PAD, CNT_W), jnp.float32),
        jax.ShapeDtypeStruct((B,), jnp.int32),
    ],
    mesh=_sc_mesh,
    scratch_types=[
        pltpu.VMEM((CPW * CHUNK,), jnp.int32),
        pltpu.VMEM((NBUF, CHUNK, D), jnp.float32),
        pltpu.VMEM((CHUNK, CNT_W), jnp.float32),
        pltpu.VMEM((STRIPE, D), jnp.float32),
        pltpu.VMEM((STRIPE, CNT_W), jnp.float32),
        pltpu.VMEM((TB,), jnp.int32),
        pltpu.VMEM((TB,), jnp.int32),
        pltpu.SemaphoreType.DMA,
        pltpu.SemaphoreType.DMA,
        pltpu.SemaphoreType.DMA,
    ] + [pltpu.SemaphoreType.DMA] * NBUF
      + [pltpu.SemaphoreType.DMA] * NBUF
      + [
        pltpu.VMEM_SHARED((C_PAD, D), jnp.float32),
        pltpu.VMEM_SHARED((C_PAD, CNT_W), jnp.float32),
    ],
    compiler_params=pltpu.CompilerParams(use_tc_tiling_on_sc=False),
)
def _sc_segment_sum(feat_hbm, lbl_hbm, idxs_hbm, featout, cntout, tgtout,
                    lbl_v, feat_v, ones_v, zf_v, zc_v, tidx_v, tgt_v,
                    sem_l, sem_c, sem_t, *rest):
    sem_ld = rest[:NBUF]
    sem_sc = rest[NBUF:2 * NBUF]
    acc_f, acc_c = rest[2 * NBUF:]
    cid = lax.axis_index("c")
    sid = lax.axis_index("s")
    wid = sid * NC + cid
    iota16 = lax.iota(jnp.int32, 16)
    zero16 = jnp.zeros((16,), jnp.float32)
    one0 = jnp.where(iota16 == 0, 1.0, 0.0).astype(jnp.float32)

    def chunk_rows(j):
        return pl.ds((wid + j * NW) * CHUNK, CHUNK)

    def lbl_idx(j):
        return lbl_v.at[pl.ds(j * CHUNK, CHUNK)]

    lbl_loads = [
        pltpu.async_copy(lbl_hbm.at[chunk_rows(j)],
                         lbl_v.at[pl.ds(j * CHUNK, CHUNK)], sem_l)
        for j in range(CPW)
    ]
    load_h = [None] * CPW
    for j in range(DEPTH):
        load_h[j] = pltpu.async_copy(
            feat_hbm.at[chunk_rows(j), pl.ds(0, D)],
            feat_v.at[j % NBUF], sem_ld[j % NBUF])

    def _fill(r, carry):
        for c4 in range(D // 16):
            zf_v[r, pl.ds(c4 * 16, 16)] = zero16
        zc_v[r, pl.ds(0, 16)] = zero16
        return carry
    lax.fori_loop(0, STRIPE, _fill, 0)

    def _fill_ones(r, carry):
        ones_v[r, pl.ds(0, 16)] = one0
        return carry
    lax.fori_loop(0, CHUNK, _fill_ones, 0)

    row0 = sid * STRIPE
    pltpu.sync_copy(zf_v, acc_f.at[pl.ds(row0, STRIPE)])
    pltpu.sync_copy(zc_v, acc_c.at[pl.ds(row0, STRIPE)])
    plsc.subcore_barrier()

    tb0 = wid * TB
    pltpu.sync_copy(idxs_hbm.at[pl.ds(tb0, TB)], tidx_v)
    tgt_gather = pltpu.async_copy(lbl_hbm.at[tidx_v], tgt_v, sem_t)

    for h in lbl_loads:
        h.wait()

    scat_h = [None] * CPW
    scat_done = [False] * CPW
    cnt_h = []
    for j in range(CPW):
        load_h[j].wait()
        scat_h[j] = pltpu.async_copy(
            feat_v.at[j % NBUF], acc_f.at[lbl_idx(j)],
            sem_sc[j % NBUF], add=True)
        cnt_h.append(
            pltpu.async_copy(ones_v, acc_c.at[lbl_idx(j)], sem_c, add=True))
        nxt = j + DEPTH
        if nxt < CPW:
            prev = nxt - NBUF
            if prev >= 0:
                scat_h[prev].wait()
                scat_done[prev] = True
            load_h[nxt] = pltpu.async_copy(
                feat_hbm.at[chunk_rows(nxt), pl.ds(0, D)],
                feat_v.at[nxt % NBUF], sem_ld[nxt % NBUF])
    for j in range(CPW):
        if not scat_done[j]:
            scat_h[j].wait()
    for h in cnt_h:
        h.wait()

    tgt_gather.wait()
    pltpu.sync_copy(tgt_v, tgtout.at[pl.ds(tb0, TB)])

    plsc.subcore_barrier()
    pltpu.sync_copy(acc_f.at[pl.ds(row0, STRIPE)],
                    featout.at[cid, pl.ds(row0, STRIPE)])
    pltpu.sync_copy(acc_c.at[pl.ds(row0, STRIPE)],
                    cntout.at[cid, pl.ds(row0, STRIPE)])


def _tc_body(x_ref, fp_ref, cp_ref, t_ref, out_ref):
    i = pl.program_id(0)
    fp = fp_ref[...]
    cs = fp[0] + fp[1]
    cp = cp_ref[...]
    cnt = jnp.sum(cp[0] + cp[1], axis=1, keepdims=True)
    rowid = lax.broadcasted_iota(jnp.int32, (C_PAD, 1), 0)
    nonempty = cnt > 0.0
    valid = jnp.logical_and(nonempty, rowid < C)
    denom = jnp.where(nonempty, cnt, 1.0)
    csw = cs * ((1.0 / TEMP) / denom)
    x = x_ref[...]
    vt = lax.dot_general(csw, x, (((1,), (1,)), ((), ())),
                         preferred_element_type=jnp.float32)
    et = jnp.exp(vt) * valid.astype(jnp.float32)
    s = jnp.sum(et, axis=0, keepdims=True)
    t = t_ref[0]
    cid2 = lax.broadcasted_iota(jnp.int32, (C_PAD, BB), 0)
    oh = (cid2 == t).astype(jnp.float32)
    val_t = jnp.sum(et * oh, axis=0, keepdims=True)
    lp = jnp.log(val_t / (s + 1e-6) + 1e-6)
    part = jnp.sum(lp, axis=1, keepdims=True)

    @pl.when(i == 0)
    def _():
        out_ref[...] = jnp.zeros_like(out_ref)

    out_ref[...] += part

    @pl.when(i == GRID - 1)
    def _():
        out_ref[...] = out_ref[...] * (-1.0 / B)


def _tc_loss(x, fp, cp, t3):
    return pl.pallas_call(
        _tc_body,
        grid=(GRID,),
        in_specs=[
            pl.BlockSpec((BB, D), lambda i: (i, 0)),
            pl.BlockSpec((NC, C_PAD, D), lambda i: (0, 0, 0)),
            pl.BlockSpec((NC, C_PAD, CNT_W), lambda i: (0, 0, 0)),
            pl.BlockSpec((1, 1, BB), lambda i: (i, 0, 0)),
        ],
        out_specs=pl.BlockSpec((1, 1), lambda i: (0, 0)),
        out_shape=jax.ShapeDtypeStruct((1, 1), jnp.float32),
    )(x, fp, cp, t3)


def kernel(inputs, indexes, features, labels, domain=0):
    featp = jnp.pad(features, ((0, NP - N), (0, DP - D)))
    lblp = jnp.pad(labels, (0, NP - N), constant_values=C)
    feat_p, cnt_p, targets = _sc_segment_sum(featp, lblp, indexes)
    t3 = jnp.reshape(targets, (GRID, 1, BB))
    loss = _tc_loss(inputs, feat_p, cnt_p, t3)
    return jnp.reshape(loss, ())
